# baseline (speedup 1.0000x reference)
import functools

import jax
import jax.numpy as jnp
from jax import lax
from jax.experimental import pallas as pl
from jax.experimental.pallas import tpu as pltpu
from jax.experimental.pallas import tpu_sc as plsc

VOCAB = 1000000
EMBED_DIM = 32
BATCH = 4096
HIST = 200

NUM_CORES = 2
NUM_SUBCORES = 16
NW = NUM_CORES * NUM_SUBCORES
HT = BATCH // (128 * NW) * (HIST // 8)
NUNIT = HIST

_mesh = plsc.VectorSubcoreMesh(core_axis_name="c", subcore_axis_name="s")


@functools.partial(
    pl.kernel,
    mesh=_mesh,
    out_type=jax.ShapeDtypeStruct((HIST, 4, NW, 8, 128), jnp.float32),
    scratch_types=[
        pltpu.VMEM((HIST // 8, 8, 128), jnp.int32),
        [pltpu.VMEM((128, EMBED_DIM), jnp.float32) for _ in range(2)],
        pltpu.VMEM((4, 8, 128), jnp.float32),
        [pltpu.SemaphoreType.DMA for _ in range(2)],
    ],
    compiler_params=pltpu.CompilerParams(
        use_tc_tiling_on_sc=False, needs_layout_passes=False
    ),
)
def _gather_kernel(xp_hbm, table_hbm, out_hbm, idx_all, rows, outblk, sems):
    bt = lax.axis_index("s") * NUM_CORES + lax.axis_index("c")

    pltpu.sync_copy(xp_hbm.at[:, bt], idx_all)

    def issue(u, b):
        idx_list = idx_all.at[u // 8, u % 8]
        pltpu.async_copy(table_hbm.at[idx_list], rows[b], sems[b])

    def process(u, b):
        idx_list = idx_all.at[u // 8, u % 8]
        pltpu.make_async_copy(table_hbm.at[idx_list], rows[b], sems[b]).wait()
        rowbase = [lax.iota(jnp.int32, 16) + blk * 16 for blk in range(8)]
        coli = jnp.zeros((16,), jnp.int32)
        for d in range(EMBED_DIM):
            for blk in range(8):
                vals = plsc.load_gather(rows[b], [rowbase[blk], coli])
                outblk[d // 8, d % 8, pl.ds(blk * 16, 16)] = vals
            coli = coli + 1
        pltpu.sync_copy(outblk, out_hbm.at[u, :, bt])

    issue(0, 0)

    def pair(p, carry):
        for j in range(2):
            u = p * 2 + j

            @pl.when(u + 1 < NUNIT)
            def _():
                issue(u + 1, 1 - j)

            process(u, j)
        return carry

    lax.fori_loop(0, NUNIT // 2, pair, 0)


def kernel(x, vectors):
    xp = x.T.reshape(HIST // 8, 8, NW, 128).transpose(0, 2, 1, 3)
    out5 = _gather_kernel(xp, vectors)
    return out5.transpose(2, 4, 0, 1, 3).reshape(BATCH, HIST, EMBED_DIM)


# speedup vs baseline: 1.2508x; 1.2508x over previous
import functools

import jax
import jax.numpy as jnp
from jax import lax
from jax.experimental import pallas as pl
from jax.experimental.pallas import tpu as pltpu
from jax.experimental.pallas import tpu_sc as plsc

VOCAB = 1000000
EMBED_DIM = 32
BATCH = 4096
HIST = 200

NUM_CORES = 2
NUM_SUBCORES = 16
NW = NUM_CORES * NUM_SUBCORES
TOTAL = BATCH * HIST
PER_W = TOTAL // NW
CHUNK = 1280
NCH = PER_W // CHUNK

_mesh = plsc.VectorSubcoreMesh(core_axis_name="c", subcore_axis_name="s")


@functools.partial(
    pl.kernel,
    mesh=_mesh,
    out_type=jax.ShapeDtypeStruct((NW, PER_W, EMBED_DIM), jnp.float32),
    scratch_types=[
        pltpu.VMEM((PER_W,), jnp.int32),
        [pltpu.VMEM((CHUNK, EMBED_DIM), jnp.float32) for _ in range(2)],
        [pltpu.SemaphoreType.DMA for _ in range(2)],
        [pltpu.SemaphoreType.DMA for _ in range(2)],
    ],
    compiler_params=pltpu.CompilerParams(
        use_tc_tiling_on_sc=False, needs_layout_passes=False
    ),
)
def _gather_kernel(x_hbm, table_hbm, out_hbm, idx, rows, gsems, wsems):
    w = lax.axis_index("s") * NUM_CORES + lax.axis_index("c")

    pltpu.sync_copy(x_hbm.at[w], idx)

    def issue_gather(u, b):
        idx_list = idx.at[pl.ds(u * CHUNK, CHUNK)]
        pltpu.async_copy(table_hbm.at[idx_list], rows[b], gsems[b])

    def issue_write(u, b):
        pltpu.async_copy(rows[b], out_hbm.at[w, pl.ds(u * CHUNK, CHUNK)],
                         wsems[b])

    issue_gather(0, 0)

    def pair(p, carry):
        for j in range(2):
            u = p * 2 + j
            pltpu.make_async_copy(
                table_hbm.at[idx.at[pl.ds(u * CHUNK, CHUNK)]],
                rows[j], gsems[j]).wait()
            issue_write(u, j)
            @pl.when(u + 1 < NCH)
            def _():
                @pl.when(u >= 1)
                def _():
                    pltpu.make_async_copy(
                        rows[1 - j],
                        out_hbm.at[w, pl.ds((u - 1) * CHUNK, CHUNK)],
                        wsems[1 - j]).wait()
                issue_gather(u + 1, 1 - j)
        return carry

    lax.fori_loop(0, NCH // 2, pair, 0)

    pltpu.make_async_copy(
        rows[0], out_hbm.at[w, pl.ds((NCH - 2) * CHUNK, CHUNK)],
        wsems[0]).wait()
    pltpu.make_async_copy(
        rows[1], out_hbm.at[w, pl.ds((NCH - 1) * CHUNK, CHUNK)],
        wsems[1]).wait()


def kernel(x, vectors):
    xw = x.reshape(NW, PER_W)
    out = _gather_kernel(xw, vectors)
    return out.reshape(BATCH, HIST, EMBED_DIM)
